# Initial kernel scaffold; baseline (speedup 1.0000x reference)
import jax
import jax.numpy as jnp
from jax.experimental import pallas as pl


def kernel(x, positions, weights):
    raise NotImplementedError("write your pallas kernel here")


# speedup vs baseline: 2.3876x; 2.3876x over previous
import functools

import jax
import jax.numpy as jnp
from jax import lax
from jax.experimental import pallas as pl
from jax.experimental.pallas import tpu as pltpu
from jax.experimental.pallas import tpu_sc as plsc


def _make_gather(num_rows, dim, total, num_cores, num_subcores):
    nw = num_cores * num_subcores
    bpw = total // nw
    chunk = 32
    nch = bpw // chunk
    assert total % nw == 0 and bpw % chunk == 0 and nch % 2 == 0 and nch >= 4

    mesh = plsc.VectorSubcoreMesh(core_axis_name="c", subcore_axis_name="s")

    @functools.partial(
        pl.kernel,
        out_type=jax.ShapeDtypeStruct((total, dim), jnp.float32),
        mesh=mesh,
        scratch_types=[
            pltpu.VMEM((bpw,), jnp.int32),
            pltpu.VMEM((chunk, dim), jnp.float32),
            pltpu.VMEM((chunk, dim), jnp.float32),
            pltpu.SemaphoreType.DMA,
            pltpu.SemaphoreType.DMA,
            pltpu.SemaphoreType.DMA,
            pltpu.SemaphoreType.DMA,
        ],
    )
    def gather_kernel(tbl, pos, out, idx_v, buf0, buf1, g0, g1, o0, o1):
        wid = lax.axis_index("s") * num_cores + lax.axis_index("c")
        base = wid * bpw
        pltpu.sync_copy(pos.at[pl.ds(base, bpw)], idx_v)

        bufs = (buf0, buf1)
        gsems = (g0, g1)
        osems = (o0, o1)

        def gather_desc(i, b):
            return pltpu.make_async_copy(
                tbl.at[idx_v.at[pl.ds(i * chunk, chunk)]], bufs[b], gsems[b])

        def out_desc(i, b):
            return pltpu.make_async_copy(
                bufs[b], out.at[pl.ds(base + i * chunk, chunk)], osems[b])

        gather_desc(0, 0).start()
        gather_desc(1, 1).start()

        def pair(p, carry):
            for b in range(2):
                i = p * 2 + b
                gather_desc(i, b).wait()
                out_desc(i, b).start()
                out_desc(i, b).wait()
                gather_desc(i + 2, b).start()
            return carry

        lax.fori_loop(0, nch // 2 - 1, pair, 0, unroll=False)

        for b in range(2):
            i = nch - 2 + b
            gather_desc(i, b).wait()
            out_desc(i, b).start()
            out_desc(i, b).wait()

    return gather_kernel


def kernel(x, positions, weights):
    bsz, seq_len = positions.shape
    num_rows, dim = weights.shape
    total = bsz * seq_len
    info = plsc.get_sparse_core_info()
    fn = _make_gather(num_rows, dim, total, info.num_cores, info.num_subcores)
    out = fn(weights, positions.reshape(total))
    return out.reshape(bsz, seq_len, dim)
